# Initial kernel scaffold; baseline (speedup 1.0000x reference)
import jax
import jax.numpy as jnp
from jax.experimental import pallas as pl


def kernel(img, col):
    raise NotImplementedError("write your pallas kernel here")


# speedup vs baseline: 2.7836x; 2.7836x over previous
import jax
import jax.numpy as jnp
from jax.experimental import pallas as pl
from jax.experimental.pallas import tpu as pltpu

_T = 0.25
_H = 2048
_W = 2048
_BLK = 256


def _body(col_ref, img_ref, out_ref):
    r = img_ref[0]
    g = img_ref[1]
    b = img_ref[2]
    hit = jnp.logical_and(jnp.logical_and(r < _T, g < _T), b < _T)
    nrm = jnp.sqrt(r * r + g * g + b * b)
    a = jnp.minimum(nrm, _T) * (1.0 / _T)
    cr = col_ref[0]
    cg = col_ref[1]
    cb = col_ref[2]
    out_ref[0] = jnp.where(hit, a * (r - cr) + cr, r)
    out_ref[1] = jnp.where(hit, a * (g - cg) + cg, g)
    out_ref[2] = jnp.where(hit, a * (b - cb) + cb, b)


def kernel(img, col):
    grid = (_H // _BLK,)
    return pl.pallas_call(
        _body,
        grid=grid,
        in_specs=[
            pl.BlockSpec(memory_space=pltpu.SMEM),
            pl.BlockSpec((3, _BLK, _W), lambda i: (0, i, 0)),
        ],
        out_specs=pl.BlockSpec((3, _BLK, _W), lambda i: (0, i, 0)),
        out_shape=jax.ShapeDtypeStruct((3, _H, _W), jnp.float32),
    )(col.reshape(3), img)
